# baseline (speedup 1.0000x reference)
import functools

import jax
import jax.numpy as jnp
from jax import lax
from jax.experimental import pallas as pl
from jax.experimental.pallas import tpu as pltpu
from jax.experimental.pallas import tpu_sc as plsc

VOCAB = 100000
HIDDEN = 1024
SEQ = 2048

_NC = 2
_NS = 16
_NW = _NC * _NS
_ROWS_PER_TILE = SEQ // _NW


def _sc_gather(table, ids):
    mesh = plsc.VectorSubcoreMesh(core_axis_name="c", subcore_axis_name="s")

    @functools.partial(
        pl.kernel,
        mesh=mesh,
        out_type=jax.ShapeDtypeStruct((SEQ, HIDDEN), jnp.float32),
        scratch_types=[
            pltpu.VMEM((_ROWS_PER_TILE,), jnp.int32),
            pltpu.VMEM((_ROWS_PER_TILE, HIDDEN), jnp.float32),
            pltpu.SemaphoreType.DMA,
        ],
    )
    def k(table_hbm, idx_hbm, out_hbm, idx_v, rows_v, sem):
        wid = lax.axis_index("s") * _NC + lax.axis_index("c")
        base = wid * _ROWS_PER_TILE
        pltpu.sync_copy(idx_hbm.at[pl.ds(base, _ROWS_PER_TILE)], idx_v)
        pltpu.async_copy(table_hbm.at[idx_v], rows_v, sem).wait()
        pltpu.sync_copy(rows_v, out_hbm.at[pl.ds(base, _ROWS_PER_TILE)])

    return k(table, ids)


_BV = 512
_NBLK = (VOCAB + _BV - 1) // _BV


def _proj_body(h_ref, w_ref, b_ref, out_ref):
    acc = lax.dot_general(
        h_ref[...], w_ref[...],
        (((1,), (1,)), ((), ())),
        preferred_element_type=jnp.float32,
    )
    out_ref[...] = acc + b_ref[...]


def _tc_project(hidden, W_out, b2d):
    return pl.pallas_call(
        _proj_body,
        grid=(_NBLK,),
        in_specs=[
            pl.BlockSpec((SEQ, HIDDEN), lambda i: (0, 0)),
            pl.BlockSpec((_BV, HIDDEN), lambda i: (i, 0)),
            pl.BlockSpec((1, _BV), lambda i: (0, i)),
        ],
        out_specs=pl.BlockSpec((SEQ, _BV), lambda i: (0, i)),
        out_shape=jax.ShapeDtypeStruct((SEQ, VOCAB), jnp.float32),
    )(hidden, W_out, b2d)


def kernel(input_ids, embedding_table, W_out, b_out):
    ids = input_ids.reshape(SEQ).astype(jnp.int32)
    hidden = _sc_gather(embedding_table, ids)
    logits = _tc_project(hidden, W_out, b_out.reshape(1, VOCAB))
    return (logits.reshape(1, SEQ, VOCAB), hidden.reshape(1, SEQ, HIDDEN))
